# baseline (speedup 1.0000x reference)
import functools

import jax
import jax.numpy as jnp
from jax import lax
from jax.experimental import pallas as pl
from jax.experimental.pallas import tpu as pltpu
from jax.experimental.pallas import tpu_sc as plsc

B = 16384
M = 32
WIDE = 128
RPW_TBL = WIDE // M
NC = 2
NS = 16
NW = NC * NS
BPW = B // NW
CH = 128
NCH = BPW // CH
IDX_ROWS = B // CH
VQ = 1000000 // RPW_TBL


def _sc_gather_emb(uq2d, iq2d, uemb_w, iemb_w):
  mesh = plsc.VectorSubcoreMesh(core_axis_name="c", subcore_axis_name="s")
  f32 = jnp.float32
  out_type = (
      jax.ShapeDtypeStruct((B, WIDE), f32),
      jax.ShapeDtypeStruct((B, WIDE), f32),
  )

  @functools.partial(
      pl.kernel,
      out_type=out_type,
      mesh=mesh,
      scratch_types=[
          pltpu.VMEM((NCH, CH), jnp.int32),
          pltpu.VMEM((NCH, CH), jnp.int32),
          pltpu.VMEM((CH, WIDE), f32),
          pltpu.VMEM((CH, WIDE), f32),
          pltpu.VMEM((CH, WIDE), f32),
          pltpu.VMEM((CH, WIDE), f32),
          pltpu.SemaphoreType.DMA,
      ],
  )
  def k(uq_hbm, iq_hbm, uemb_hbm, iemb_hbm, u_out, v_out,
        uq_v, iq_v, ubuf0, ubuf1, vbuf0, vbuf1, sem):
    wid = lax.axis_index("s") * NC + lax.axis_index("c")
    rowbase = wid * NCH
    pltpu.sync_copy(uq_hbm.at[pl.ds(rowbase, NCH)], uq_v)
    pltpu.sync_copy(iq_hbm.at[pl.ds(rowbase, NCH)], iq_v)
    base = wid * BPW
    ubufs = (ubuf0, ubuf1)
    vbufs = (vbuf0, vbuf1)
    hs = [None, None]
    for j in range(NCH):
      p = j % 2
      if hs[p] is not None:
        hu, hv, row = hs[p]
        hu.wait()
        hv.wait()
        pltpu.sync_copy(ubufs[p], u_out.at[pl.ds(row, CH)])
        pltpu.sync_copy(vbufs[p], v_out.at[pl.ds(row, CH)])
      hu = pltpu.async_copy(uemb_hbm.at[uq_v.at[j]], ubufs[p], sem)
      hv = pltpu.async_copy(iemb_hbm.at[iq_v.at[j]], vbufs[p], sem)
      hs[p] = (hu, hv, base + j * CH)
    for p in range(2):
      hu, hv, row = hs[p]
      hu.wait()
      hv.wait()
      pltpu.sync_copy(ubufs[p], u_out.at[pl.ds(row, CH)])
      pltpu.sync_copy(vbufs[p], v_out.at[pl.ds(row, CH)])

  return k(uq2d, iq2d, uemb_w, iemb_w)


def _sc_gather_bias(user2d, item2d, user_bias, item_bias):
  mesh = plsc.VectorSubcoreMesh(core_axis_name="c", subcore_axis_name="s")
  f32 = jnp.float32
  out_type = (
      jax.ShapeDtypeStruct((IDX_ROWS, CH), f32),
      jax.ShapeDtypeStruct((IDX_ROWS, CH), f32),
  )

  @functools.partial(
      pl.kernel,
      out_type=out_type,
      mesh=mesh,
      compiler_params=pltpu.CompilerParams(use_tc_tiling_on_sc=False),
      scratch_types=[
          pltpu.VMEM((NCH, CH), jnp.int32),
          pltpu.VMEM((NCH, CH), jnp.int32),
          pltpu.VMEM((NCH, CH), f32),
          pltpu.VMEM((NCH, CH), f32),
          pltpu.SemaphoreType.DMA,
      ],
  )
  def k(user_hbm, item_hbm, ubias_hbm, ibias_hbm, ub_out, ib_out,
        uidx_v, iidx_v, ub_v, ib_v, sem):
    wid = lax.axis_index("s") * NC + lax.axis_index("c")
    rowbase = wid * NCH
    pltpu.sync_copy(user_hbm.at[pl.ds(rowbase, NCH)], uidx_v)
    pltpu.sync_copy(item_hbm.at[pl.ds(rowbase, NCH)], iidx_v)
    copies = []
    for j in range(NCH):
      copies.append(pltpu.async_copy(
          ubias_hbm.at[uidx_v.at[j]], ub_v.at[j], sem))
      copies.append(pltpu.async_copy(
          ibias_hbm.at[iidx_v.at[j]], ib_v.at[j], sem))
    for c in copies:
      c.wait()
    pltpu.sync_copy(ub_v, ub_out.at[pl.ds(rowbase, NCH)])
    pltpu.sync_copy(ib_v, ib_out.at[pl.ds(rowbase, NCH)])

  return k(user2d, item2d, user_bias, item_bias)


TC_BLK = 2048


def _tc_dot(u4, v4, ur, ir, ub, ib, offset):
  def body(u4_ref, v4_ref, ur_ref, ir_ref, ub_ref, ib_ref, off_ref, o_ref):
    urc = ur_ref[...][:, None]
    irc = ir_ref[...][:, None]
    usel = jnp.zeros((TC_BLK, M), jnp.float32)
    vsel = jnp.zeros((TC_BLK, M), jnp.float32)
    for kk in range(RPW_TBL):
      slab = slice(kk * M, (kk + 1) * M)
      usel = usel + jnp.where(urc == kk, u4_ref[:, slab], 0.0)
      vsel = vsel + jnp.where(irc == kk, v4_ref[:, slab], 0.0)
    prod = jnp.sum(usel * vsel, axis=1)
    o_ref[...] = prod + ub_ref[...] + ib_ref[...] + off_ref[...]

  grid = (B // TC_BLK,)
  return pl.pallas_call(
      body,
      grid=grid,
      in_specs=[
          pl.BlockSpec((TC_BLK, WIDE), lambda i: (i, 0)),
          pl.BlockSpec((TC_BLK, WIDE), lambda i: (i, 0)),
          pl.BlockSpec((TC_BLK,), lambda i: (i,)),
          pl.BlockSpec((TC_BLK,), lambda i: (i,)),
          pl.BlockSpec((TC_BLK,), lambda i: (i,)),
          pl.BlockSpec((TC_BLK,), lambda i: (i,)),
          pl.BlockSpec((1,), lambda i: (0,)),
      ],
      out_specs=pl.BlockSpec((TC_BLK,), lambda i: (i,)),
      out_shape=jax.ShapeDtypeStruct((B,), jnp.float32),
  )(u4, v4, ur, ir, ub, ib, offset)


@jax.jit
def kernel(user, item, user_emb, item_emb, user_bias, item_bias, offset):
  user = user.astype(jnp.int32)
  item = item.astype(jnp.int32)
  uq2d = (user // RPW_TBL).reshape(IDX_ROWS, CH)
  iq2d = (item // RPW_TBL).reshape(IDX_ROWS, CH)
  uemb_w = user_emb.reshape(VQ, WIDE)
  iemb_w = item_emb.reshape(VQ, WIDE)
  u4, v4 = _sc_gather_emb(uq2d, iq2d, uemb_w, iemb_w)
  ub2, ib2 = _sc_gather_bias(
      user.reshape(IDX_ROWS, CH), item.reshape(IDX_ROWS, CH),
      user_bias, item_bias)
  return _tc_dot(u4, v4, user % RPW_TBL, item % RPW_TBL,
                 ub2.reshape(B), ib2.reshape(B), offset)
